# baseline (speedup 1.0000x reference)
import functools

import jax
import jax.numpy as jnp
import numpy as np
from jax import lax
from jax.experimental import pallas as pl
from jax.experimental.pallas import tpu as pltpu
from jax.experimental.pallas import tpu_sc as plsc

INPUT_DIM = 4096
K = 1024
TEMP = 5.0
ROWS = 4 * 2048
BLK = 512
NCHUNK = INPUT_DIM // 16
EXT = 256

def _soft_probs(logits):
    u = jnp.clip(
        jax.random.uniform(jax.random.key(1), logits.shape, dtype=jnp.float32),
        1e-06,
        None,
    )
    noise = -jnp.log(-jnp.log(u) + 1e-06)
    return jax.nn.sigmoid((logits + noise) / TEMP)


def _sc_kth_kernel(sp_hbm, out_hbm, sp_v, ext_v, out_v):
    cid = lax.axis_index("c")
    sid = lax.axis_index("s")

    @pl.when(jnp.logical_and(cid == 0, sid == 0))
    def _():
        pltpu.sync_copy(sp_hbm, sp_v)
        k_v = jnp.full((16,), K, jnp.int32)

        def count_ge(t_v):
            def body(i, acc):
                return acc + plsc.all_reduce_population_count(
                    sp_v[pl.ds(i * 16, 16)] >= t_v
                )

            return lax.fori_loop(0, NCHUNK, body, jnp.zeros((16,), jnp.int32))

        def nbody(_, lohi):
            lo, hi = lohi
            mid = 0.5 * (lo + hi)
            big = count_ge(mid) >= k_v
            return (jnp.where(big, mid, lo), jnp.where(big, hi, mid))

        lo, hi = lax.fori_loop(
            0, 8, nbody,
            (jnp.zeros((16,), jnp.float32), jnp.ones((16,), jnp.float32)),
        )
        r_v = k_v - count_ge(hi)

        for i in range(EXT // 16):
            ext_v[pl.ds(i * 16, 16)] = jnp.full((16,), -1.0, jnp.float32)

        ones_i = jnp.ones((16,), jnp.int32)
        zeros_i = jnp.zeros((16,), jnp.int32)

        def ebody(i, off):
            v = sp_v[pl.ds(i * 16, 16)]
            m = jnp.logical_and(v >= lo, v < hi)
            mi = jnp.where(m, ones_i, zeros_i)
            excl = plsc.cumsum(mi) - mi
            idx = jnp.minimum(off, EXT - 48) + excl
            plsc.store_scatter(ext_v, [idx], v, mask=m)
            return off + plsc.all_reduce_population_count(m)

        lax.fori_loop(0, NCHUNK, ebody, jnp.zeros((16,), jnp.int32))

        def kbody(_, bounds):
            blo, bhi = bounds
            mid = blo + (bhi - blo + 1) // 2
            t_v = plsc.bitcast(mid, jnp.float32)

            def cbody(i, acc):
                return acc + plsc.all_reduce_population_count(
                    ext_v[pl.ds(i * 16, 16)] >= t_v
                )

            acc = lax.fori_loop(
                0, EXT // 16, cbody, jnp.zeros((16,), jnp.int32)
            )
            big = acc >= r_v
            return (jnp.where(big, mid, blo), jnp.where(big, bhi, mid - 1))

        blo = plsc.bitcast(lo, jnp.int32)
        bhi = plsc.bitcast(hi, jnp.int32)
        blo, bhi = lax.fori_loop(0, 31, kbody, (blo, bhi))

        out_v[...] = plsc.bitcast(blo, jnp.float32)
        pltpu.sync_copy(out_v, out_hbm)


_sc_kth = pl.kernel(
    _sc_kth_kernel,
    out_type=jax.ShapeDtypeStruct((16,), jnp.float32),
    compiler_params=pltpu.CompilerParams(needs_layout_passes=False),
    mesh=plsc.VectorSubcoreMesh(core_axis_name="c", subcore_axis_name="s"),
    scratch_types=[
        pltpu.VMEM((INPUT_DIM,), jnp.float32),
        pltpu.VMEM((EXT,), jnp.float32),
        pltpu.VMEM((16,), jnp.float32),
    ],
)


def _mask_mul_kernel(x_ref, sp_ref, kth_ref, o_ref):
    mask = (sp_ref[...] >= kth_ref[0, 0]).astype(jnp.float32)
    o_ref[...] = x_ref[...] * mask


@jax.jit
def kernel(x, logits):
    sp = _soft_probs(logits)

    kth16 = _sc_kth(sp)

    x2d = x.reshape(ROWS, INPUT_DIM)
    out = pl.pallas_call(
        _mask_mul_kernel,
        grid=(ROWS // BLK,),
        in_specs=[
            pl.BlockSpec((BLK, INPUT_DIM), lambda i: (i, 0)),
            pl.BlockSpec((1, INPUT_DIM), lambda i: (0, 0)),
            pl.BlockSpec((1, 16), lambda i: (0, 0)),
        ],
        out_specs=pl.BlockSpec((BLK, INPUT_DIM), lambda i: (i, 0)),
        out_shape=jax.ShapeDtypeStruct((ROWS, INPUT_DIM), jnp.float32),
        compiler_params=pltpu.CompilerParams(
            dimension_semantics=("arbitrary",),
        ),
    )(x2d, sp.reshape(1, INPUT_DIM), kth16.reshape(1, 16))
    return out.reshape(x.shape)


# speedup vs baseline: 1.0890x; 1.0890x over previous
import functools

import jax
import jax.numpy as jnp
import numpy as np
from jax import lax
from jax.experimental import pallas as pl
from jax.experimental.pallas import tpu as pltpu
from jax.experimental.pallas import tpu_sc as plsc

INPUT_DIM = 4096
K = 1024
TEMP = 5.0
ROWS = 4 * 2048
BLK = 512
NCHUNK = INPUT_DIM // 16
EXT = 512

def _soft_probs(logits):
    u = jnp.clip(
        jax.random.uniform(jax.random.key(1), logits.shape, dtype=jnp.float32),
        1e-06,
        None,
    )
    noise = -jnp.log(-jnp.log(u) + 1e-06)
    return jax.nn.sigmoid((logits + noise) / TEMP)


def _sc_kth_kernel(sp_hbm, out_hbm, sp_v, ext_v, out_v):
    cid = lax.axis_index("c")
    sid = lax.axis_index("s")

    @pl.when(jnp.logical_and(cid == 0, sid == 0))
    def _():
        pltpu.sync_copy(sp_hbm, sp_v)
        k_v = jnp.full((16,), K, jnp.int32)

        UNROLL = 8

        def count_ge(t_v):
            def body(i, acc):
                for u in range(UNROLL):
                    acc = acc + plsc.all_reduce_population_count(
                        sp_v[pl.ds((i * UNROLL + u) * 16, 16)] >= t_v
                    )
                return acc

            return lax.fori_loop(
                0, NCHUNK // UNROLL, body, jnp.zeros((16,), jnp.int32)
            )

        def nbody(_, carry):
            lo, hi, cnt_hi = carry
            mid = 0.5 * (lo + hi)
            cnt = count_ge(mid)
            big = cnt >= k_v
            return (
                jnp.where(big, mid, lo),
                jnp.where(big, hi, mid),
                jnp.where(big, cnt_hi, cnt),
            )

        lo, hi, cnt_hi = lax.fori_loop(
            0, 6, nbody,
            (
                jnp.zeros((16,), jnp.float32),
                jnp.ones((16,), jnp.float32),
                jnp.zeros((16,), jnp.int32),
            ),
        )
        r_v = k_v - cnt_hi

        for i in range(EXT // 16):
            ext_v[pl.ds(i * 16, 16)] = jnp.full((16,), -1.0, jnp.float32)

        ones_i = jnp.ones((16,), jnp.int32)
        zeros_i = jnp.zeros((16,), jnp.int32)

        def ebody(i, off):
            for u in range(UNROLL):
                v = sp_v[pl.ds((i * UNROLL + u) * 16, 16)]
                m = jnp.logical_and(v >= lo, v < hi)
                mi = jnp.where(m, ones_i, zeros_i)
                excl = plsc.cumsum(mi) - mi
                idx = jnp.minimum(off, EXT - 48) + excl
                plsc.store_scatter(ext_v, [idx], v, mask=m)
                off = off + plsc.all_reduce_population_count(m)
            return off

        lax.fori_loop(0, NCHUNK // UNROLL, ebody, jnp.zeros((16,), jnp.int32))

        def kcond(bounds):
            blo, bhi = bounds
            return jnp.any(blo < bhi)

        def kbody(bounds):
            blo, bhi = bounds
            mid = blo + (bhi - blo + 1) // 2
            t_v = plsc.bitcast(mid, jnp.float32)

            def cbody(i, acc):
                for u in range(UNROLL):
                    acc = acc + plsc.all_reduce_population_count(
                        ext_v[pl.ds((i * UNROLL + u) * 16, 16)] >= t_v
                    )
                return acc

            acc = lax.fori_loop(
                0, EXT // 16 // UNROLL, cbody, jnp.zeros((16,), jnp.int32)
            )
            big = acc >= r_v
            return (jnp.where(big, mid, blo), jnp.where(big, bhi, mid - 1))

        blo = plsc.bitcast(lo, jnp.int32)
        bhi = plsc.bitcast(hi, jnp.int32)
        blo, bhi = lax.while_loop(kcond, kbody, (blo, bhi))

        out_v[...] = plsc.bitcast(blo, jnp.float32)
        pltpu.sync_copy(out_v, out_hbm)


_sc_kth = pl.kernel(
    _sc_kth_kernel,
    out_type=jax.ShapeDtypeStruct((16,), jnp.float32),
    compiler_params=pltpu.CompilerParams(needs_layout_passes=False),
    mesh=plsc.VectorSubcoreMesh(core_axis_name="c", subcore_axis_name="s"),
    scratch_types=[
        pltpu.VMEM((INPUT_DIM,), jnp.float32),
        pltpu.VMEM((EXT,), jnp.float32),
        pltpu.VMEM((16,), jnp.float32),
    ],
)


def _mask_mul_kernel(x_ref, sp_ref, kth_ref, o_ref):
    mask = (sp_ref[...] >= kth_ref[0, 0]).astype(jnp.float32)
    o_ref[...] = x_ref[...] * mask


@jax.jit
def kernel(x, logits):
    sp = _soft_probs(logits)

    kth16 = _sc_kth(sp)

    x2d = x.reshape(ROWS, INPUT_DIM)
    out = pl.pallas_call(
        _mask_mul_kernel,
        grid=(ROWS // BLK,),
        in_specs=[
            pl.BlockSpec((BLK, INPUT_DIM), lambda i: (i, 0)),
            pl.BlockSpec((1, INPUT_DIM), lambda i: (0, 0)),
            pl.BlockSpec((1, 16), lambda i: (0, 0)),
        ],
        out_specs=pl.BlockSpec((BLK, INPUT_DIM), lambda i: (i, 0)),
        out_shape=jax.ShapeDtypeStruct((ROWS, INPUT_DIM), jnp.float32),
        compiler_params=pltpu.CompilerParams(
            dimension_semantics=("arbitrary",),
        ),
    )(x2d, sp.reshape(1, INPUT_DIM), kth16.reshape(1, 16))
    return out.reshape(x.shape)
